# Initial kernel scaffold; baseline (speedup 1.0000x reference)
import functools

import jax
import jax.numpy as jnp
from jax import lax
from jax.experimental import pallas as pl
from jax.experimental.pallas import tpu as pltpu
from jax.experimental.pallas import tpu_sc as plsc

N = 10000
E = 320000
D = 128

NC = 2
NS = 16
NW = NC * NS

N_PAD = 10240
ROWS_PER_TILE = N_PAD // NS
CHUNK = 128
CHUNKS_PER_TILE = 79
E_PAD = NW * CHUNKS_PER_TILE * CHUNK
EDGES_PER_TILE = CHUNKS_PER_TILE * CHUNK
DUMMY = N


def _mm_body(x_ref, w_ref, ws_ref, xw_ref, xws_ref):
    xb = x_ref[...]
    xw_ref[...] = jnp.dot(xb, w_ref[...], preferred_element_type=jnp.float32)
    xws_ref[...] = jnp.dot(xb, ws_ref[...], preferred_element_type=jnp.float32)


def _matmuls(x_pad, W, W_self):
    blk = 2048
    grid = (N_PAD // blk,)
    return pl.pallas_call(
        _mm_body,
        grid=grid,
        in_specs=[
            pl.BlockSpec((blk, D), lambda i: (i, 0)),
            pl.BlockSpec((D, D), lambda i: (0, 0)),
            pl.BlockSpec((D, D), lambda i: (0, 0)),
        ],
        out_specs=[
            pl.BlockSpec((blk, D), lambda i: (i, 0)),
            pl.BlockSpec((blk, D), lambda i: (i, 0)),
        ],
        out_shape=[
            jax.ShapeDtypeStruct((N_PAD, D), jnp.float32),
            jax.ShapeDtypeStruct((N_PAD, D), jnp.float32),
        ],
    )(x_pad, W, W_self)


def _scatter_body(xw_hbm, xws_hbm, src_hbm, dst_hbm, parts_hbm,
                  src_idx, dst_idx, rows, vbuf, acc, sem):
    c = lax.axis_index("c")
    s = lax.axis_index("s")
    wid = c * NS + s

    pltpu.sync_copy(xws_hbm.at[pl.ds(s * ROWS_PER_TILE, ROWS_PER_TILE)], vbuf)
    pltpu.sync_copy(vbuf, acc.at[pl.ds(s * ROWS_PER_TILE, ROWS_PER_TILE)])

    pltpu.sync_copy(src_hbm.at[pl.ds(wid * CHUNKS_PER_TILE, CHUNKS_PER_TILE)],
                    src_idx)
    pltpu.sync_copy(dst_hbm.at[pl.ds(wid * CHUNKS_PER_TILE, CHUNKS_PER_TILE)],
                    dst_idx)
    plsc.subcore_barrier()

    def chunk(j, carry):
        pltpu.async_copy(xw_hbm.at[src_idx.at[j]], rows, sem).wait()
        pltpu.sync_copy(rows, acc.at[dst_idx.at[j]], add=True)
        return carry

    lax.fori_loop(0, CHUNKS_PER_TILE, chunk, 0)
    plsc.subcore_barrier()

    pltpu.sync_copy(acc.at[pl.ds(s * ROWS_PER_TILE, ROWS_PER_TILE)], vbuf)
    pltpu.sync_copy(vbuf,
                    parts_hbm.at[c].at[pl.ds(s * ROWS_PER_TILE, ROWS_PER_TILE)])


def _scatter_phase(xw, xws, src2d, dst2d):
    mesh = plsc.VectorSubcoreMesh(core_axis_name="c", subcore_axis_name="s",
                                  num_cores=NC, num_subcores=NS)
    fn = pl.kernel(
        _scatter_body,
        out_type=jax.ShapeDtypeStruct((NC, N_PAD, D), jnp.float32),
        mesh=mesh,
        scratch_types=[
            pltpu.VMEM((CHUNKS_PER_TILE, CHUNK), jnp.int32),
            pltpu.VMEM((CHUNKS_PER_TILE, CHUNK), jnp.int32),
            pltpu.VMEM((CHUNK, D), jnp.float32),
            pltpu.VMEM((ROWS_PER_TILE, D), jnp.float32),
            pltpu.VMEM_SHARED((N_PAD, D), jnp.float32),
            pltpu.SemaphoreType.DMA,
        ],
    )
    return fn(xw, xws, src2d, dst2d)


def _comb_body(p_ref, xws_ref, h_ref):
    h_ref[...] = jnp.maximum(p_ref[0] + p_ref[1] - xws_ref[...], 0.0)


def _combine(parts, xws):
    blk = 2048
    return pl.pallas_call(
        _comb_body,
        grid=(N_PAD // blk,),
        in_specs=[
            pl.BlockSpec((NC, blk, D), lambda i: (0, i, 0)),
            pl.BlockSpec((blk, D), lambda i: (i, 0)),
        ],
        out_specs=pl.BlockSpec((blk, D), lambda i: (i, 0)),
        out_shape=jax.ShapeDtypeStruct((N_PAD, D), jnp.float32),
    )(parts, xws)


def _dot_body(h_hbm, a_hbm, b_hbm, pred_hbm,
              a_idx, b_idx, rows_a, rows_b, out_v, sem_a, sem_b):
    c = lax.axis_index("c")
    s = lax.axis_index("s")
    wid = c * NS + s

    pltpu.sync_copy(a_hbm.at[pl.ds(wid * CHUNKS_PER_TILE, CHUNKS_PER_TILE)],
                    a_idx)
    pltpu.sync_copy(b_hbm.at[pl.ds(wid * CHUNKS_PER_TILE, CHUNKS_PER_TILE)],
                    b_idx)

    def chunk(j, carry):
        cp_a = pltpu.async_copy(h_hbm.at[a_idx.at[j]], rows_a, sem_a)
        cp_b = pltpu.async_copy(h_hbm.at[b_idx.at[j]], rows_b, sem_b)
        cp_a.wait()
        cp_b.wait()

        def row(r, carry2):
            acc = (rows_a[r, pl.ds(0, 16)] * rows_b[r, pl.ds(0, 16)])
            for t in range(1, D // 16):
                acc = acc + (rows_a[r, pl.ds(16 * t, 16)] *
                             rows_b[r, pl.ds(16 * t, 16)])
            out_v[j * CHUNK + r] = jnp.sum(acc)
            return carry2

        lax.fori_loop(0, CHUNK, row, 0)
        return carry

    lax.fori_loop(0, CHUNKS_PER_TILE, chunk, 0)
    pltpu.sync_copy(out_v, pred_hbm.at[pl.ds(wid * EDGES_PER_TILE,
                                             EDGES_PER_TILE)])


def _dot_phase(h, a2d, b2d):
    mesh = plsc.VectorSubcoreMesh(core_axis_name="c", subcore_axis_name="s",
                                  num_cores=NC, num_subcores=NS)
    fn = pl.kernel(
        _dot_body,
        out_type=jax.ShapeDtypeStruct((E_PAD,), jnp.float32),
        mesh=mesh,
        scratch_types=[
            pltpu.VMEM((CHUNKS_PER_TILE, CHUNK), jnp.int32),
            pltpu.VMEM((CHUNKS_PER_TILE, CHUNK), jnp.int32),
            pltpu.VMEM((CHUNK, D), jnp.float32),
            pltpu.VMEM((CHUNK, D), jnp.float32),
            pltpu.VMEM((EDGES_PER_TILE,), jnp.float32),
            pltpu.SemaphoreType.DMA,
            pltpu.SemaphoreType.DMA,
        ],
    )
    return fn(h, a2d, b2d)


def _pad_idx(idx):
    pad = jnp.full((E_PAD - E,), DUMMY, dtype=idx.dtype)
    return jnp.concatenate([idx, pad]).reshape(NW * CHUNKS_PER_TILE, CHUNK)


@jax.jit
def kernel(x, edge_index, edge_label_index, W, W_self):
    x_pad = jnp.concatenate(
        [x, jnp.zeros((N_PAD - N, D), dtype=x.dtype)], axis=0)
    xw, xws = _matmuls(x_pad, W, W_self)

    src2d = _pad_idx(edge_index[0])
    dst2d = _pad_idx(edge_index[1])
    parts = _scatter_phase(xw, xws, src2d, dst2d)
    h = _combine(parts, xws)

    a2d = _pad_idx(edge_label_index[0])
    b2d = _pad_idx(edge_label_index[1])
    pred_pad = _dot_phase(h, a2d, b2d)
    return pred_pad[:E]


# speedup vs baseline: 1.3085x; 1.3085x over previous
import functools

import jax
import jax.numpy as jnp
from jax import lax
from jax.experimental import pallas as pl
from jax.experimental.pallas import tpu as pltpu
from jax.experimental.pallas import tpu_sc as plsc

N = 10000
E = 320000
D = 128

NC = 2
NS = 16
NW = NC * NS

N_PAD = 10240
ROWS_PER_TILE = N_PAD // NS
CHUNK = 128
CHUNKS_PER_TILE = 80
E_PAD = NW * CHUNKS_PER_TILE * CHUNK
EDGES_PER_TILE = CHUNKS_PER_TILE * CHUNK
DUMMY = N


def _mm_body(x_ref, w_ref, ws_ref, xw_ref, xws_ref):
    xb = x_ref[...]
    xw_ref[...] = jnp.dot(xb, w_ref[...], preferred_element_type=jnp.float32)
    xws_ref[...] = jnp.dot(xb, ws_ref[...], preferred_element_type=jnp.float32)


def _matmuls(x_pad, W, W_self):
    blk = 2048
    grid = (N_PAD // blk,)
    return pl.pallas_call(
        _mm_body,
        grid=grid,
        in_specs=[
            pl.BlockSpec((blk, D), lambda i: (i, 0)),
            pl.BlockSpec((D, D), lambda i: (0, 0)),
            pl.BlockSpec((D, D), lambda i: (0, 0)),
        ],
        out_specs=[
            pl.BlockSpec((blk, D), lambda i: (i, 0)),
            pl.BlockSpec((blk, D), lambda i: (i, 0)),
        ],
        out_shape=[
            jax.ShapeDtypeStruct((N_PAD, D), jnp.float32),
            jax.ShapeDtypeStruct((N_PAD, D), jnp.float32),
        ],
    )(x_pad, W, W_self)


def _scatter_body(xw_hbm, xws_hbm, src_hbm, dst_hbm, parts_hbm,
                  src_idx, dst_idx, rows, acc, sem):
    c = lax.axis_index("c")
    s = lax.axis_index("s")
    wid = c * NS + s

    pltpu.sync_copy(xws_hbm.at[pl.ds(s * ROWS_PER_TILE, ROWS_PER_TILE)],
                    acc.at[pl.ds(s * ROWS_PER_TILE, ROWS_PER_TILE)])

    pltpu.sync_copy(src_hbm.at[pl.ds(wid * CHUNKS_PER_TILE, CHUNKS_PER_TILE)],
                    src_idx)
    pltpu.sync_copy(dst_hbm.at[pl.ds(wid * CHUNKS_PER_TILE, CHUNKS_PER_TILE)],
                    dst_idx)
    plsc.subcore_barrier()

    def chunk(j, carry):
        pltpu.async_copy(xw_hbm.at[src_idx.at[j]], rows, sem).wait()
        pltpu.sync_copy(rows, acc.at[dst_idx.at[j]], add=True)
        return carry

    lax.fori_loop(0, CHUNKS_PER_TILE, chunk, 0)
    plsc.subcore_barrier()

    pltpu.sync_copy(acc.at[pl.ds(s * ROWS_PER_TILE, ROWS_PER_TILE)],
                    parts_hbm.at[c].at[pl.ds(s * ROWS_PER_TILE, ROWS_PER_TILE)])


def _scatter_phase(xw, xws, src2d, dst2d):
    mesh = plsc.VectorSubcoreMesh(core_axis_name="c", subcore_axis_name="s",
                                  num_cores=NC, num_subcores=NS)
    fn = pl.kernel(
        _scatter_body,
        out_type=jax.ShapeDtypeStruct((NC, N_PAD, D), jnp.float32),
        mesh=mesh,
        scratch_types=[
            pltpu.VMEM((CHUNKS_PER_TILE, CHUNK), jnp.int32),
            pltpu.VMEM((CHUNKS_PER_TILE, CHUNK), jnp.int32),
            pltpu.VMEM((CHUNK, D), jnp.float32),
            pltpu.VMEM_SHARED((N_PAD, D), jnp.float32),
            pltpu.SemaphoreType.DMA,
        ],
    )
    return fn(xw, xws, src2d, dst2d)


def _comb_body(p_ref, xws_ref, h_ref):
    h_ref[...] = jnp.maximum(p_ref[0] + p_ref[1] - xws_ref[...], 0.0)


def _combine(parts, xws):
    blk = 2048
    return pl.pallas_call(
        _comb_body,
        grid=(N_PAD // blk,),
        in_specs=[
            pl.BlockSpec((NC, blk, D), lambda i: (0, i, 0)),
            pl.BlockSpec((blk, D), lambda i: (i, 0)),
        ],
        out_specs=pl.BlockSpec((blk, D), lambda i: (i, 0)),
        out_shape=jax.ShapeDtypeStruct((N_PAD, D), jnp.float32),
    )(parts, xws)


def _dot_body(h_hbm, a_hbm, b_hbm, pred_hbm,
              a_idx, b_idx, rows_a, rows_b, out_v, sem_a, sem_b):
    c = lax.axis_index("c")
    s = lax.axis_index("s")
    wid = c * NS + s

    pltpu.sync_copy(a_hbm.at[pl.ds(wid * CHUNKS_PER_TILE, CHUNKS_PER_TILE)],
                    a_idx)
    pltpu.sync_copy(b_hbm.at[pl.ds(wid * CHUNKS_PER_TILE, CHUNKS_PER_TILE)],
                    b_idx)

    lane = lax.iota(jnp.int32, 16)

    def chunk(j, carry):
        cp_a = pltpu.async_copy(h_hbm.at[a_idx.at[j]], rows_a, sem_a)
        cp_b = pltpu.async_copy(h_hbm.at[b_idx.at[j]], rows_b, sem_b)
        cp_a.wait()
        cp_b.wait()

        def group(g, carry2):
            row_ids = g * 16 + lane

            def dcol(d, acc):
                col = jnp.full((16,), 0, jnp.int32) + d
                va = plsc.load_gather(rows_a, [row_ids, col])
                vb = plsc.load_gather(rows_b, [row_ids, col])
                return acc + va * vb

            dots = lax.fori_loop(0, D, dcol, jnp.zeros((16,), jnp.float32),
                                 unroll=8)
            out_v[pl.ds(j * CHUNK + g * 16, 16)] = dots
            return carry2

        lax.fori_loop(0, CHUNK // 16, group, 0)
        return carry

    lax.fori_loop(0, CHUNKS_PER_TILE, chunk, 0)
    pltpu.sync_copy(out_v, pred_hbm.at[pl.ds(wid * EDGES_PER_TILE,
                                             EDGES_PER_TILE)])


def _dot_phase(h, a2d, b2d):
    mesh = plsc.VectorSubcoreMesh(core_axis_name="c", subcore_axis_name="s",
                                  num_cores=NC, num_subcores=NS)
    fn = pl.kernel(
        _dot_body,
        out_type=jax.ShapeDtypeStruct((E_PAD,), jnp.float32),
        mesh=mesh,
        scratch_types=[
            pltpu.VMEM((CHUNKS_PER_TILE, CHUNK), jnp.int32),
            pltpu.VMEM((CHUNKS_PER_TILE, CHUNK), jnp.int32),
            pltpu.VMEM((CHUNK, D), jnp.float32),
            pltpu.VMEM((CHUNK, D), jnp.float32),
            pltpu.VMEM((EDGES_PER_TILE,), jnp.float32),
            pltpu.SemaphoreType.DMA,
            pltpu.SemaphoreType.DMA,
        ],
        compiler_params=pltpu.CompilerParams(needs_layout_passes=False),
    )
    return fn(h, a2d, b2d)


def _pad_idx(idx):
    pad = jnp.full((E_PAD - E,), DUMMY, dtype=idx.dtype)
    return jnp.concatenate([idx, pad]).reshape(NW * CHUNKS_PER_TILE, CHUNK)


@jax.jit
def kernel(x, edge_index, edge_label_index, W, W_self):
    x_pad = jnp.concatenate(
        [x, jnp.zeros((N_PAD - N, D), dtype=x.dtype)], axis=0)
    xw, xws = _matmuls(x_pad, W, W_self)

    src2d = _pad_idx(edge_index[0])
    dst2d = _pad_idx(edge_index[1])
    parts = _scatter_phase(xw, xws, src2d, dst2d)
    h = _combine(parts, xws)

    a2d = _pad_idx(edge_label_index[0])
    b2d = _pad_idx(edge_label_index[1])
    pred_pad = _dot_phase(h, a2d, b2d)
    return pred_pad[:E]


# speedup vs baseline: 2.4767x; 1.8928x over previous
import functools

import jax
import jax.numpy as jnp
from jax import lax
from jax.experimental import pallas as pl
from jax.experimental.pallas import tpu as pltpu
from jax.experimental.pallas import tpu_sc as plsc

N = 10000
E = 320000
D = 128

NC = 2
NS = 16
NW = NC * NS

N_PAD = 10240
ROWS_PER_TILE = N_PAD // NS
CHUNK = 128
CHUNKS_PER_TILE = 80
E_PAD = NW * CHUNKS_PER_TILE * CHUNK
EDGES_PER_TILE = CHUNKS_PER_TILE * CHUNK
DUMMY = N


def _mm_body(x_ref, w_ref, ws_ref, xw_ref, xws_ref):
    xb = x_ref[...]
    xw_ref[...] = jnp.dot(xb, w_ref[...], preferred_element_type=jnp.float32)
    xws_ref[...] = jnp.dot(xb, ws_ref[...], preferred_element_type=jnp.float32)


def _matmuls(x_pad, W, W_self):
    blk = 2048
    grid = (N_PAD // blk,)
    return pl.pallas_call(
        _mm_body,
        grid=grid,
        in_specs=[
            pl.BlockSpec((blk, D), lambda i: (i, 0)),
            pl.BlockSpec((D, D), lambda i: (0, 0)),
            pl.BlockSpec((D, D), lambda i: (0, 0)),
        ],
        out_specs=[
            pl.BlockSpec((blk, D), lambda i: (i, 0)),
            pl.BlockSpec((blk, D), lambda i: (i, 0)),
        ],
        out_shape=[
            jax.ShapeDtypeStruct((N_PAD, D), jnp.float32),
            jax.ShapeDtypeStruct((N_PAD, D), jnp.float32),
        ],
    )(x_pad, W, W_self)


DST_BLK = 8


def _scatter_body(xw_hbm, xws_hbm, src_hbm, dst_hbm, parts_hbm,
                  src_idx, dst_idx, rows0, rows1, acc, sem0, sem1):
    c = lax.axis_index("c")
    s = lax.axis_index("s")
    wid = c * NS + s

    pltpu.sync_copy(xws_hbm.at[pl.ds(s * ROWS_PER_TILE, ROWS_PER_TILE)],
                    acc.at[pl.ds(s * ROWS_PER_TILE, ROWS_PER_TILE)])

    pltpu.sync_copy(src_hbm.at[pl.ds(wid * CHUNKS_PER_TILE, CHUNKS_PER_TILE)],
                    src_idx)
    plsc.subcore_barrier()

    bufs = (rows0, rows1)
    sems = (sem0, sem1)

    pltpu.async_copy(xw_hbm.at[src_idx.at[0]], rows0, sem0)

    def group(grp, carry):
        pltpu.sync_copy(
            dst_hbm.at[pl.ds(wid * CHUNKS_PER_TILE + grp * DST_BLK, DST_BLK)],
            dst_idx)
        for i in range(DST_BLK):
            j = grp * DST_BLK + i

            @pl.when(j + 1 < CHUNKS_PER_TILE)
            def _fire():
                pltpu.async_copy(xw_hbm.at[src_idx.at[j + 1]],
                                 bufs[(i + 1) % 2], sems[(i + 1) % 2])

            pltpu.make_async_copy(xw_hbm.at[src_idx.at[j]], bufs[i % 2],
                                  sems[i % 2]).wait()
            pltpu.sync_copy(bufs[i % 2], acc.at[dst_idx.at[i]], add=True)
        return carry

    lax.fori_loop(0, CHUNKS_PER_TILE // DST_BLK, group, 0)
    plsc.subcore_barrier()

    pltpu.sync_copy(acc.at[pl.ds(s * ROWS_PER_TILE, ROWS_PER_TILE)],
                    parts_hbm.at[c].at[pl.ds(s * ROWS_PER_TILE, ROWS_PER_TILE)])


def _scatter_phase(xw, xws, src2d, dst2d):
    mesh = plsc.VectorSubcoreMesh(core_axis_name="c", subcore_axis_name="s",
                                  num_cores=NC, num_subcores=NS)
    fn = pl.kernel(
        _scatter_body,
        out_type=jax.ShapeDtypeStruct((NC, N_PAD, D), jnp.float32),
        mesh=mesh,
        scratch_types=[
            pltpu.VMEM((CHUNKS_PER_TILE, CHUNK), jnp.int32),
            pltpu.VMEM((DST_BLK, CHUNK), jnp.int32),
            pltpu.VMEM((CHUNK, D), jnp.float32),
            pltpu.VMEM((CHUNK, D), jnp.float32),
            pltpu.VMEM_SHARED((N_PAD, D), jnp.float32),
            pltpu.SemaphoreType.DMA,
            pltpu.SemaphoreType.DMA,
        ],
    )
    return fn(xw, xws, src2d, dst2d)


def _comb_body(p_ref, xws_ref, h_ref):
    h_ref[...] = jnp.maximum(p_ref[0] + p_ref[1] - xws_ref[...], 0.0)


def _combine(parts, xws):
    blk = 2048
    return pl.pallas_call(
        _comb_body,
        grid=(N_PAD // blk,),
        in_specs=[
            pl.BlockSpec((NC, blk, D), lambda i: (0, i, 0)),
            pl.BlockSpec((blk, D), lambda i: (i, 0)),
        ],
        out_specs=pl.BlockSpec((blk, D), lambda i: (i, 0)),
        out_shape=jax.ShapeDtypeStruct((N_PAD, D), jnp.float32),
    )(parts, xws)


def _dot_chunk_compute(ra, rb, part, out_v, out_base, lane):
    for g in range(CHUNK // 16):
        for r in range(16):
            row = g * 16 + r
            acc = ra[row, pl.ds(0, 16)] * rb[row, pl.ds(0, 16)]
            for t in range(1, D // 16):
                acc = acc + (ra[row, pl.ds(16 * t, 16)] *
                             rb[row, pl.ds(16 * t, 16)])
            part[r, :] = acc
        dots = jnp.zeros((16,), jnp.float32)
        for t in range(16):
            col = jnp.full((16,), t, jnp.int32)
            dots = dots + plsc.load_gather(part, [lane, col])
        out_v[pl.ds(out_base + g * 16, 16)] = dots


def _dot_body(h_hbm, a_hbm, b_hbm, pred_hbm,
              a_idx, b_idx, ra0, rb0, ra1, rb1, part, out_v,
              sa0, sb0, sa1, sb1):
    c = lax.axis_index("c")
    s = lax.axis_index("s")
    wid = c * NS + s

    pltpu.sync_copy(a_hbm.at[pl.ds(wid * CHUNKS_PER_TILE, CHUNKS_PER_TILE)],
                    a_idx)
    pltpu.sync_copy(b_hbm.at[pl.ds(wid * CHUNKS_PER_TILE, CHUNKS_PER_TILE)],
                    b_idx)

    lane = lax.iota(jnp.int32, 16)
    ra = (ra0, ra1)
    rb = (rb0, rb1)
    sa = (sa0, sa1)
    sb = (sb0, sb1)

    pltpu.async_copy(h_hbm.at[a_idx.at[0]], ra0, sa0)
    pltpu.async_copy(h_hbm.at[b_idx.at[0]], rb0, sb0)

    def pair(p, carry):
        for i in range(2):
            j = 2 * p + i

            @pl.when(j + 1 < CHUNKS_PER_TILE)
            def _fire():
                pltpu.async_copy(h_hbm.at[a_idx.at[j + 1]], ra[(i + 1) % 2],
                                 sa[(i + 1) % 2])
                pltpu.async_copy(h_hbm.at[b_idx.at[j + 1]], rb[(i + 1) % 2],
                                 sb[(i + 1) % 2])

            pltpu.make_async_copy(h_hbm.at[a_idx.at[j]], ra[i], sa[i]).wait()
            pltpu.make_async_copy(h_hbm.at[b_idx.at[j]], rb[i], sb[i]).wait()
            _dot_chunk_compute(ra[i], rb[i], part, out_v, j * CHUNK, lane)
        return carry

    lax.fori_loop(0, CHUNKS_PER_TILE // 2, pair, 0)
    pltpu.sync_copy(out_v, pred_hbm.at[pl.ds(wid * EDGES_PER_TILE,
                                             EDGES_PER_TILE)])


def _dot_phase(h, a2d, b2d):
    mesh = plsc.VectorSubcoreMesh(core_axis_name="c", subcore_axis_name="s",
                                  num_cores=NC, num_subcores=NS)
    fn = pl.kernel(
        _dot_body,
        out_type=jax.ShapeDtypeStruct((E_PAD,), jnp.float32),
        mesh=mesh,
        scratch_types=[
            pltpu.VMEM((CHUNKS_PER_TILE, CHUNK), jnp.int32),
            pltpu.VMEM((CHUNKS_PER_TILE, CHUNK), jnp.int32),
            pltpu.VMEM((CHUNK, D), jnp.float32),
            pltpu.VMEM((CHUNK, D), jnp.float32),
            pltpu.VMEM((CHUNK, D), jnp.float32),
            pltpu.VMEM((CHUNK, D), jnp.float32),
            pltpu.VMEM((16, 16), jnp.float32),
            pltpu.VMEM((EDGES_PER_TILE,), jnp.float32),
            pltpu.SemaphoreType.DMA,
            pltpu.SemaphoreType.DMA,
            pltpu.SemaphoreType.DMA,
            pltpu.SemaphoreType.DMA,
        ],
        compiler_params=pltpu.CompilerParams(needs_layout_passes=False),
    )
    return fn(h, a2d, b2d)


def _pad_idx(idx):
    pad = jnp.full((E_PAD - E,), DUMMY, dtype=idx.dtype)
    return jnp.concatenate([idx, pad]).reshape(NW * CHUNKS_PER_TILE, CHUNK)


@jax.jit
def kernel(x, edge_index, edge_label_index, W, W_self):
    x_pad = jnp.concatenate(
        [x, jnp.zeros((N_PAD - N, D), dtype=x.dtype)], axis=0)
    xw, xws = _matmuls(x_pad, W, W_self)

    src2d = _pad_idx(edge_index[0])
    dst2d = _pad_idx(edge_index[1])
    parts = _scatter_phase(xw, xws, src2d, dst2d)
    h = _combine(parts, xws)

    a2d = _pad_idx(edge_label_index[0])
    b2d = _pad_idx(edge_label_index[1])
    pred_pad = _dot_phase(h, a2d, b2d)
    return pred_pad[:E]
